# baseline (speedup 1.0000x reference)
import numpy as np
import jax
import jax.numpy as jnp
from jax.experimental import pallas as pl
from jax.experimental.pallas import tpu as pltpu

_B = 32
_R = 100
_IND = 64
_K1 = 50
_K2 = 25
_NC = 2
_KCOM = 8
_F = 64


def _dg(a, b, lc, rc, prec=jax.lax.Precision.DEFAULT):
    return jax.lax.dot_general(
        a, b, (((lc,), (rc,)), ((), ())),
        precision=prec,
        preferred_element_type=jnp.float32)


def _bdg(a, b, lc, rc):
    return _dg(a.astype(jnp.bfloat16), b.astype(jnp.bfloat16), lc, rc)


def _b32(a):
    return a.astype(jnp.bfloat16).astype(jnp.float32)


_HI = jax.lax.Precision.HIGHEST


def _eye(n):
    r = jax.lax.broadcasted_iota(jnp.int32, (n, n), 0)
    c = jax.lax.broadcasted_iota(jnp.int32, (n, n), 1)
    return r == c


def _gin_layer(A, pos_g, x_g, k, weights, w_pre):
    (nnW1, W2, Bb, eps, Wu1, bu1, Wu2, bu2, wpool, Wq, bq, Wk, bk) = weights
    Rc = A.shape[0]
    eye = _eye(Rc)
    eyef = eye.astype(jnp.float32)
    mask = A > 0.0
    mask_loc = jnp.logical_or(mask, eye)
    A_loc = jnp.where(jnp.logical_and(eye, jnp.logical_not(mask)), 1.0, A)
    if w_pre is None:
        h = jnp.maximum(_bdg(pos_g, nnW1, 1, 0), 0.0)
        w = _bdg(h, W2, 1, 0)
        xf = _b32(x_g)
        sl = lambda i: _b32(w[:, i * _F:(i + 1) * _F] + Bb[i:i + 1, :])
    else:
        xf = x_g
        sl = lambda i: w_pre[:, i * _F:(i + 1) * _F]
    xt = xf[:, 0:1] * sl(0)
    for i in range(1, _IND):
        xt = xt + xf[:, i:i + 1] * sl(i)
    logits = jnp.where(mask_loc, A_loc, -jnp.inf)
    p = jax.nn.softmax(logits, axis=-1)
    aggr = _bdg(p, xt, 1, 0)
    upd = (1.0 + eps) * xt + aggr
    hid = jnp.maximum(_bdg(upd, Wu1, 1, 0) + bu1, 0.0)
    out = _bdg(hid, Wu2, 1, 0) + bu2
    nrm = jnp.sqrt(jnp.sum(wpool * wpool))
    sc = jax.nn.sigmoid(
        jnp.sum(_b32(out) * _b32(wpool), axis=1, keepdims=True) / nrm)
    sc_row = _dg(sc, eyef, 0, 0, _HI)
    gt = (sc_row > sc).astype(jnp.float32)
    ri = jax.lax.broadcasted_iota(jnp.int32, (Rc, Rc), 0)
    ci = jax.lax.broadcasted_iota(jnp.int32, (Rc, Rc), 1)
    tie = jnp.logical_and(sc_row == sc, ci < ri).astype(jnp.float32)
    rank = jnp.sum(gt + tie, axis=1, keepdims=True)
    rank_row = _dg(rank, eyef, 0, 0, _HI)
    jr = jax.lax.broadcasted_iota(jnp.int32, (k, Rc), 0)
    P = (jr == rank_row.astype(jnp.int32)).astype(jnp.float32)
    vals = _dg(P, sc, 1, 0, _HI)
    vals_row = _dg(sc, P, 0, 1, _HI)
    x_new = _dg(P, out, 1, 0, _HI) * vals
    A_new = _dg(_dg(P, A, 1, 0, _HI), P, 1, 1, _HI)
    pos_new = _dg(P, pos_g, 1, 0, _HI)
    xmean = jnp.sum(x_new, axis=0, keepdims=True) / float(k)
    xq = _bdg(xmean, Wq, 1, 0) + bq
    xk_ = _bdg(x_new, Wk, 1, 0) + bk
    att = jax.nn.sigmoid(jnp.sum(_b32(xk_) * _b32(xq), axis=1,
                                 keepdims=True) / np.sqrt(float(_F)))
    garo = jnp.sum(x_new * att, axis=0, keepdims=True) / float(k)
    eyek = _eye(k).astype(jnp.float32)
    Ai = A_new + eyek
    A2 = _bdg(Ai, Ai, 1, 0) * (1.0 - eyek)
    return garo, vals_row, A2, pos_new, x_new


def _body(A_ref, pos_ref, x_ref, w1_ref,
          nnW1_ref, M_ref, Bb_ref, eps_ref, Wu1_ref, bu1_ref, Wu2_ref,
          bu2_ref, wpool_ref, Wq_ref, bq_ref, Wk_ref, bk_ref,
          Wf1_ref, bf1_ref, g1_ref, be1_ref, Wf2_ref, bf2_ref, g2_ref,
          be2_ref, Wfin_ref, bfin_ref,
          out_ref, s1_ref, s2_ref):
    A = A_ref[0]
    pos_g = pos_ref[0]
    x_g = x_ref[0]
    garos = []
    svals = []
    ks = (_K1, _K2)
    for li in range(2):
        weights = (nnW1_ref[li], M_ref[li], Bb_ref[li], eps_ref[li],
                   Wu1_ref[li], bu1_ref[li], Wu2_ref[li], bu2_ref[li],
                   wpool_ref[li], Wq_ref[li], bq_ref[li], Wk_ref[li],
                   bk_ref[li])
        garo, vals_row, A, pos_g, x_g = _gin_layer(
            A, pos_g, x_g, ks[li], weights,
            w1_ref[0] if li == 0 else None)
        garos.append(garo)
        svals.append(vals_row)
    hfc = jnp.concatenate(garos, axis=1)
    for (W, b, g, be) in ((Wf1_ref, bf1_ref, g1_ref, be1_ref),
                          (Wf2_ref, bf2_ref, g2_ref, be2_ref)):
        hfc = jnp.maximum(_bdg(hfc, W[...], 1, 0) + b[...], 0.0)
        hfc = hfc / np.sqrt(1.0 + 1e-5) * g[...] + be[...]
    out_ref[0] = jnp.maximum(_bdg(hfc, Wfin_ref[...], 1, 0) + bfin_ref[...],
                             0.0)
    s1_ref[0] = svals[0]
    s2_ref[0] = svals[1]


@jax.jit
def _run(A, posb, xb, w1, *w):
    in_specs = [
        pl.BlockSpec((1, _R, _R), lambda b: (b, 0, 0)),
        pl.BlockSpec((1, _R, _R), lambda b: (b, 0, 0)),
        pl.BlockSpec((1, _R, _IND), lambda b: (b, 0, 0)),
        pl.BlockSpec((1, _R, _IND * _F), lambda b: (b, 0, 0)),
    ]
    for wi in w:
        in_specs.append(
            pl.BlockSpec(wi.shape, lambda b, n=wi.ndim: (0,) * n))
    out_shape = [jax.ShapeDtypeStruct((_B, 1, _NC), jnp.float32),
                 jax.ShapeDtypeStruct((_B, 1, _K1), jnp.float32),
                 jax.ShapeDtypeStruct((_B, 1, _K2), jnp.float32)]
    out_specs = [pl.BlockSpec((1, 1, _NC), lambda b: (b, 0, 0)),
                 pl.BlockSpec((1, 1, _K1), lambda b: (b, 0, 0)),
                 pl.BlockSpec((1, 1, _K2), lambda b: (b, 0, 0))]
    out, s1, s2 = pl.pallas_call(
        _body, grid=(_B,), in_specs=in_specs, out_specs=out_specs,
        out_shape=out_shape)(A, posb, xb, w1, *w)
    return out[:, 0], s1[:, 0], s2[:, 0]


def kernel(x, edge_index, batch, edge_attr, pos, params):
    src = edge_index[0]
    dst = edge_index[1]
    g = src // _R
    s = src % _R
    d = dst % _R
    A = jnp.zeros((_B, _R, _R), jnp.float32).at[g, d, s].add(edge_attr)
    xb = x.reshape(_B, _R, _IND)
    posb = pos.reshape(_B, _R, _R)
    L = params['layers']

    def st(name):
        return jnp.stack([lp[name] for lp in L])

    nnW1 = st('nnW1')
    M = st('nnW2')
    Bb = st('nnb2').reshape(2, _F, _F)
    eps = st('eps').reshape(2, 1, 1)
    Wu1 = st('Wu1')
    bu1 = st('bu1').reshape(2, 1, 2 * _F)
    Wu2 = st('Wu2')
    bu2 = st('bu2').reshape(2, 1, _F)
    wpool = st('wpool').reshape(2, 1, _F)
    Wq = st('Wq')
    bq = st('bq').reshape(2, 1, _F)
    Wk = st('Wk')
    bk = st('bk').reshape(2, 1, _F)
    f1, f2 = params['fcs']
    l0 = L[0]
    h0 = jax.nn.relu(jnp.einsum('bnr,rk->bnk', posb, l0['nnW1']))
    w1 = jnp.einsum('bnk,ko->bno', h0, l0['nnW2']) + l0['nnb2']
    out, s1, s2 = _run(
        A, posb, xb, w1, nnW1, M, Bb, eps, Wu1, bu1, Wu2, bu2, wpool, Wq, bq,
        Wk, bk,
        f1['W'], f1['b'].reshape(1, -1), f1['gamma'].reshape(1, -1),
        f1['beta'].reshape(1, -1),
        f2['W'], f2['b'].reshape(1, -1), f2['gamma'].reshape(1, -1),
        f2['beta'].reshape(1, -1),
        params['Wfin'], params['bfin'].reshape(1, -1))
    return (out, s1, s2)
